# Initial kernel scaffold; baseline (speedup 1.0000x reference)
import jax
import jax.numpy as jnp
from jax.experimental import pallas as pl


def kernel(x, topk_ids, topk_weights, w13_qweight, w13_scales, w2_qweight, w2_scales):
    raise NotImplementedError("write your pallas kernel here")


# speedup vs baseline: 2.5940x; 2.5940x over previous
import jax
import jax.numpy as jnp
from jax.experimental import pallas as pl
from jax.experimental.pallas import tpu as pltpu

_E = 8
_D = 1024
_F = 2048
_T = 128
_GROUP = 128
_FB = 512


def _moe_kernel(ids_ref, tw_ref, x_ref, wg_ref, sg_ref, wu_ref, su_ref,
                w2_ref, s2_ref, out_ref):
  e = pl.program_id(0)
  j = pl.program_id(1)

  shifts = (jnp.arange(8, dtype=jnp.int32) * 4)[None, :, None]

  def dequant(q, s, k8, groups):
    n = q.shape[-1]
    nib = ((q[:, None, :] >> shifts) & 0xF).astype(jnp.float32)
    srep = jnp.broadcast_to(s[:, None, :], (groups, _GROUP // 8, n))
    srep = srep.reshape(k8, n)
    w = (nib - 8.0) * srep[:, None, :]
    return w.reshape(k8 * 8, n).astype(jnp.bfloat16)

  x = x_ref[...]
  wg = dequant(wg_ref[0], sg_ref[0], _D // 8, _D // _GROUP)
  hg = jnp.dot(x, wg, preferred_element_type=jnp.float32)
  wu = dequant(wu_ref[0], su_ref[0], _D // 8, _D // _GROUP)
  hu = jnp.dot(x, wu, preferred_element_type=jnp.float32)

  we = jnp.sum(jnp.where(ids_ref[...] == e, tw_ref[...], 0.0), axis=1,
               keepdims=True)
  act = (hg * jax.nn.sigmoid(hg) * hu * we).astype(jnp.bfloat16)

  w2 = dequant(w2_ref[0], s2_ref[0, 0], _FB // 8, _FB // _GROUP)
  o = jnp.dot(act, w2, preferred_element_type=jnp.float32)

  @pl.when((e == 0) & (j == 0))
  def _init():
    out_ref[...] = jnp.zeros_like(out_ref)

  out_ref[...] += o


@jax.jit
def kernel(x, topk_ids, topk_weights, w13_qweight, w13_scales, w2_qweight,
           w2_scales):
  jblocks = _F // _FB
  grid = (_E, jblocks)

  out = pl.pallas_call(
      _moe_kernel,
      grid=grid,
      in_specs=[
          pl.BlockSpec((_T, 2), lambda e, j: (0, 0)),
          pl.BlockSpec((_T, 2), lambda e, j: (0, 0)),
          pl.BlockSpec((_T, _D), lambda e, j: (0, 0)),
          pl.BlockSpec((1, _D // 8, _FB), lambda e, j: (e, 0, j)),
          pl.BlockSpec((1, _D // _GROUP, _FB), lambda e, j: (e, 0, j)),
          pl.BlockSpec((1, _D // 8, _FB), lambda e, j: (e, 0, j + jblocks)),
          pl.BlockSpec((1, _D // _GROUP, _FB), lambda e, j: (e, 0, j + jblocks)),
          pl.BlockSpec((1, _FB // 8, _D), lambda e, j: (e, j, 0)),
          pl.BlockSpec((1, 1, _FB // _GROUP, _D), lambda e, j: (e, j, 0, 0)),
      ],
      out_specs=pl.BlockSpec((_T, _D), lambda e, j: (0, 0)),
      out_shape=jax.ShapeDtypeStruct((_T, _D), jnp.float32),
      compiler_params=pltpu.CompilerParams(
          dimension_semantics=("arbitrary", "arbitrary"),
      ),
  )(
      topk_ids.astype(jnp.int32),
      topk_weights,
      x.astype(jnp.bfloat16),
      w13_qweight,
      w13_scales,
      w13_qweight,
      w13_scales,
      w2_qweight,
      w2_scales.reshape(_E, jblocks, _FB // _GROUP, _D),
  )
  return out


# speedup vs baseline: 3.2458x; 1.2513x over previous
import jax
import jax.numpy as jnp
from jax.experimental import pallas as pl
from jax.experimental.pallas import tpu as pltpu

_E = 8
_D = 1024
_F = 2048
_T = 128
_GROUP = 128
_FB = 512


def _moe_kernel(ids_ref, tw_ref, x_ref, wg_ref, sg_ref, wu_ref, su_ref,
                w2_ref, s2_ref, out_ref):
  e = pl.program_id(0)
  j = pl.program_id(1)

  shifts = (jnp.arange(8, dtype=jnp.int32) * 4)[None, :, None]

  def srep_rows(s, k8, groups):
    n = s.shape[-1]
    srep = jnp.broadcast_to(s[:, None, :], (groups, _GROUP // 8, n))
    return srep.reshape(k8, n)

  def dequant(q, s, k8, groups):
    n = q.shape[-1]
    nib = ((q[:, None, :] >> shifts) & 0xF).astype(jnp.float32)
    w = (nib - 8.0) * srep_rows(s, k8, groups)[:, None, :]
    return w.reshape(k8 * 8, n).astype(jnp.bfloat16)

  def dequant_planes(q, s, k8, groups):
    srep = srep_rows(s, k8, groups)
    planes = []
    for i in range(8):
      nib = ((q >> (4 * i)) & 0xF).astype(jnp.float32)
      planes.append((nib - 8.0) * srep)
    return jnp.concatenate(planes, axis=0).astype(jnp.bfloat16)

  x = x_ref[...]
  wg = dequant_planes(wg_ref[0], sg_ref[0], _D // 8, _D // _GROUP)
  hg = jnp.dot(x, wg, preferred_element_type=jnp.float32)
  wu = dequant_planes(wu_ref[0], su_ref[0], _D // 8, _D // _GROUP)
  hu = jnp.dot(x, wu, preferred_element_type=jnp.float32)

  we = jnp.sum(jnp.where(ids_ref[...] == e, tw_ref[...], 0.0), axis=1,
               keepdims=True)
  act = (hg * jax.nn.sigmoid(hg) * hu * we).astype(jnp.bfloat16)

  w2 = dequant(w2_ref[0], s2_ref[0, 0], _FB // 8, _FB // _GROUP)
  o = jnp.dot(act, w2, preferred_element_type=jnp.float32)

  @pl.when((e == 0) & (j == 0))
  def _init():
    out_ref[...] = jnp.zeros_like(out_ref)

  out_ref[...] += o


@jax.jit
def kernel(x, topk_ids, topk_weights, w13_qweight, w13_scales, w2_qweight,
           w2_scales):
  jblocks = _F // _FB
  grid = (_E, jblocks)

  out = pl.pallas_call(
      _moe_kernel,
      grid=grid,
      in_specs=[
          pl.BlockSpec((_T, 2), lambda e, j: (0, 0)),
          pl.BlockSpec((_T, 2), lambda e, j: (0, 0)),
          pl.BlockSpec((_T, _D), lambda e, j: (0, 0)),
          pl.BlockSpec((1, _D // 8, _FB), lambda e, j: (e, 0, j)),
          pl.BlockSpec((1, _D // _GROUP, _FB), lambda e, j: (e, 0, j)),
          pl.BlockSpec((1, _D // 8, _FB), lambda e, j: (e, 0, j + jblocks)),
          pl.BlockSpec((1, _D // _GROUP, _FB), lambda e, j: (e, 0, j + jblocks)),
          pl.BlockSpec((1, _FB // 8, _D), lambda e, j: (e, j, 0)),
          pl.BlockSpec((1, 1, _FB // _GROUP, _D), lambda e, j: (e, j, 0, 0)),
      ],
      out_specs=pl.BlockSpec((_T, _D), lambda e, j: (0, 0)),
      out_shape=jax.ShapeDtypeStruct((_T, _D), jnp.float32),
      compiler_params=pltpu.CompilerParams(
          dimension_semantics=("arbitrary", "arbitrary"),
      ),
  )(
      topk_ids.astype(jnp.int32),
      topk_weights,
      x.reshape(_T, _D // 8, 8).transpose(0, 2, 1).reshape(_T, _D)
      .astype(jnp.bfloat16),
      w13_qweight,
      w13_scales,
      w13_qweight,
      w13_scales,
      w2_qweight,
      w2_scales.reshape(_E, jblocks, _FB // _GROUP, _D),
  )
  return out


# speedup vs baseline: 3.5209x; 1.0848x over previous
import jax
import jax.numpy as jnp
from jax.experimental import pallas as pl
from jax.experimental.pallas import tpu as pltpu

_E = 8
_D = 1024
_F = 2048
_T = 128
_GROUP = 128
_FB = 512


def _moe_kernel(ids_ref, tw_ref, x_ref, p_ref, wg_ref, sg_ref, wu_ref, su_ref,
                w2_ref, s2_ref, out_ref):
  e = pl.program_id(0)
  j = pl.program_id(1)

  shifts = (jnp.arange(8, dtype=jnp.int32) * 4)[None, :, None]

  def srep_rows(s, k8, groups):
    n = s.shape[-1]
    srep = jnp.broadcast_to(s[:, None, :], (groups, _GROUP // 8, n))
    return srep.reshape(k8, n)

  def dequant(q, s, k8, groups):
    n = q.shape[-1]
    nib = ((q[:, None, :] >> shifts) & 0xF).astype(jnp.float32)
    w = (nib - 8.0) * srep_rows(s, k8, groups)[:, None, :]
    return w.reshape(k8 * 8, n).astype(jnp.bfloat16)

  def dequant_planes(q, s, k8, groups):
    srep = srep_rows(s, k8, groups)
    planes = []
    for i in range(8):
      nib = ((q >> (4 * i)) & 0xF).astype(jnp.float32)
      planes.append((nib - 8.0) * srep)
    return jnp.concatenate(planes, axis=0).astype(jnp.bfloat16)

  x = x_ref[...]
  wg = dequant_planes(wg_ref[0], sg_ref[0], _D // 8, _D // _GROUP)
  hg = jnp.dot(x, wg, preferred_element_type=jnp.float32)
  wu = dequant_planes(wu_ref[0], su_ref[0], _D // 8, _D // _GROUP)
  hu = jnp.dot(x, wu, preferred_element_type=jnp.float32)

  we = jnp.sum(jnp.where(ids_ref[...] == e, tw_ref[...], 0.0), axis=1,
               keepdims=True)
  act = (hg * jax.nn.sigmoid(hg) * hu * we).astype(jnp.bfloat16)
  act = jnp.dot(act, p_ref[...], preferred_element_type=jnp.float32)
  act = act.astype(jnp.bfloat16)

  w2 = dequant_planes(w2_ref[0], s2_ref[0, 0], _FB // 8, _FB // _GROUP)
  o = jnp.dot(act, w2, preferred_element_type=jnp.float32)

  @pl.when((e == 0) & (j == 0))
  def _init():
    out_ref[...] = jnp.zeros_like(out_ref)

  out_ref[...] += o


@jax.jit
def kernel(x, topk_ids, topk_weights, w13_qweight, w13_scales, w2_qweight,
           w2_scales):
  jblocks = _F // _FB
  grid = (_E, jblocks)

  out = pl.pallas_call(
      _moe_kernel,
      grid=grid,
      in_specs=[
          pl.BlockSpec((_T, 2), lambda e, j: (0, 0)),
          pl.BlockSpec((_T, 2), lambda e, j: (0, 0)),
          pl.BlockSpec((_T, _D), lambda e, j: (0, 0)),
          pl.BlockSpec((_FB, _FB), lambda e, j: (0, 0)),
          pl.BlockSpec((1, _D // 8, _FB), lambda e, j: (e, 0, j)),
          pl.BlockSpec((1, _D // _GROUP, _FB), lambda e, j: (e, 0, j)),
          pl.BlockSpec((1, _D // 8, _FB), lambda e, j: (e, 0, j + jblocks)),
          pl.BlockSpec((1, _D // _GROUP, _FB), lambda e, j: (e, 0, j + jblocks)),
          pl.BlockSpec((1, _FB // 8, _D), lambda e, j: (e, j, 0)),
          pl.BlockSpec((1, 1, _FB // _GROUP, _D), lambda e, j: (e, j, 0, 0)),
      ],
      out_specs=pl.BlockSpec((_T, _D), lambda e, j: (0, 0)),
      out_shape=jax.ShapeDtypeStruct((_T, _D), jnp.float32),
      compiler_params=pltpu.CompilerParams(
          dimension_semantics=("arbitrary", "arbitrary"),
      ),
  )(
      topk_ids.astype(jnp.int32),
      topk_weights,
      x.reshape(_T, _D // 8, 8).transpose(0, 2, 1).reshape(_T, _D)
      .astype(jnp.bfloat16),
      (jnp.arange(_FB)[:, None]
       == ((jnp.arange(_FB)[None, :] % (_FB // 8)) * 8
           + jnp.arange(_FB)[None, :] // (_FB // 8))).astype(jnp.bfloat16),
      w13_qweight,
      w13_scales,
      w13_qweight,
      w13_scales,
      w2_qweight,
      w2_scales.reshape(_E, jblocks, _FB // _GROUP, _D),
  )
  return out


# speedup vs baseline: 3.9398x; 1.1190x over previous
import jax
import jax.numpy as jnp
from jax.experimental import pallas as pl
from jax.experimental.pallas import tpu as pltpu

_E = 8
_D = 1024
_F = 2048
_T = 128
_GROUP = 128
_FB = 512


def _moe_kernel(ids_ref, tw_ref, x_ref, p_ref, wg_ref, sg_ref,
                wu_ref, su_ref, w2_ref, s2_ref, out_ref):
  e = pl.program_id(0)
  j = pl.program_id(1)

  shifts = (jnp.arange(8, dtype=jnp.int32) * 4)[None, :, None]

  def srep_rows(s, k8, groups):
    n = s.shape[-1]
    srep = jnp.broadcast_to(s[:, None, :], (groups, _GROUP // 8, n))
    return srep.reshape(k8, n)

  def dequant(q, s, k8, groups):
    n = q.shape[-1]
    nib = ((q[:, None, :] >> shifts) & 0xF).astype(jnp.float32)
    w = (nib - 8.0) * srep_rows(s, k8, groups)[:, None, :]
    return w.reshape(k8 * 8, n).astype(jnp.bfloat16)

  def dequant_planes(q, s, k8, groups):
    n = q.shape[-1]
    srep = srep_rows(s, k8, groups)
    sbits = ((jax.lax.bitcast_convert_type(srep, jnp.int32) + 0x8000) >> 16)
    spk = pltpu.bitcast((sbits << 16) | sbits, jnp.bfloat16)
    planes = []
    for j in range(4):
      lo = (q >> (8 * j)) & 0xF
      if j < 2:
        hi = (q << (12 - 8 * j)) & 0xF0000
      else:
        hi = (q >> (8 * j - 12)) & 0xF0000
      w = lo | hi | 0x43004300
      wb = pltpu.bitcast(w, jnp.bfloat16)
      planes.append((wb - jnp.bfloat16(136.0)) * spk)
    return jnp.concatenate(planes, axis=0)

  x = x_ref[...]
  wg = dequant_planes(wg_ref[0], sg_ref[0], _D // 8, _D // _GROUP)
  hg = jnp.dot(x, wg, preferred_element_type=jnp.float32)
  wu = dequant_planes(wu_ref[0], su_ref[0], _D // 8, _D // _GROUP)
  hu = jnp.dot(x, wu, preferred_element_type=jnp.float32)

  we = jnp.sum(jnp.where(ids_ref[...] == e, tw_ref[...], 0.0), axis=1,
               keepdims=True)
  act = (hg * jax.nn.sigmoid(hg) * hu * we).astype(jnp.bfloat16)
  act = jnp.dot(act, p_ref[...], preferred_element_type=jnp.float32)
  act = act.astype(jnp.bfloat16)

  w2 = dequant_planes(w2_ref[0], s2_ref[0, 0], _FB // 8, _FB // _GROUP)
  o = jnp.dot(act, w2, preferred_element_type=jnp.float32)

  @pl.when((e == 0) & (j == 0))
  def _init():
    out_ref[...] = jnp.zeros_like(out_ref)

  out_ref[...] += o


@jax.jit
def kernel(x, topk_ids, topk_weights, w13_qweight, w13_scales, w2_qweight,
           w2_scales):
  jblocks = _F // _FB
  grid = (_E, jblocks)

  out = pl.pallas_call(
      _moe_kernel,
      grid=grid,
      in_specs=[
          pl.BlockSpec((_T, 2), lambda e, j: (0, 0)),
          pl.BlockSpec((_T, 2), lambda e, j: (0, 0)),
          pl.BlockSpec((_T, _D), lambda e, j: (0, 0)),
          pl.BlockSpec((_FB, _FB), lambda e, j: (0, 0)),
          pl.BlockSpec((1, _D // 8, _FB), lambda e, j: (e, 0, j)),
          pl.BlockSpec((1, _D // _GROUP, _FB), lambda e, j: (e, 0, j)),
          pl.BlockSpec((1, _D // 8, _FB), lambda e, j: (e, 0, j + jblocks)),
          pl.BlockSpec((1, _D // _GROUP, _FB), lambda e, j: (e, 0, j + jblocks)),
          pl.BlockSpec((1, _FB // 8, _D), lambda e, j: (e, j, 0)),
          pl.BlockSpec((1, 1, _FB // _GROUP, _D), lambda e, j: (e, j, 0, 0)),
      ],
      out_specs=pl.BlockSpec((_T, _D), lambda e, j: (0, 0)),
      out_shape=jax.ShapeDtypeStruct((_T, _D), jnp.float32),
      compiler_params=pltpu.CompilerParams(
          dimension_semantics=("arbitrary", "arbitrary"),
      ),
  )(
      topk_ids.astype(jnp.int32),
      topk_weights,
      x.reshape(_T, _D // 8, 4, 2).transpose(0, 2, 1, 3).reshape(_T, _D)
      .astype(jnp.bfloat16),
      (jnp.arange(_FB)[:, None]
       == (((jnp.arange(_FB)[None, :] % 128) // 2) * 8
           + 2 * (jnp.arange(_FB)[None, :] // 128)
           + jnp.arange(_FB)[None, :] % 2)).astype(jnp.bfloat16),
      w13_qweight,
      w13_scales,
      w13_qweight,
      w13_scales,
      w2_qweight,
      w2_scales.reshape(_E, jblocks, _FB // _GROUP, _D),
  )
  return out


# speedup vs baseline: 4.1485x; 1.0530x over previous
import jax
import jax.numpy as jnp
from jax.experimental import pallas as pl
from jax.experimental.pallas import tpu as pltpu

_E = 8
_D = 1024
_F = 2048
_T = 128
_GROUP = 128
_FB = 1024


def _moe_kernel(ids_ref, tw_ref, x_ref, p_ref, wg_ref, sg_ref,
                wu_ref, su_ref, w2_ref, s2_ref, out_ref):
  e = pl.program_id(0)
  j = pl.program_id(1)

  shifts = (jnp.arange(8, dtype=jnp.int32) * 4)[None, :, None]

  def srep_rows(s, k8, groups):
    n = s.shape[-1]
    srep = jnp.broadcast_to(s[:, None, :], (groups, _GROUP // 8, n))
    return srep.reshape(k8, n)

  def dequant(q, s, k8, groups):
    n = q.shape[-1]
    nib = ((q[:, None, :] >> shifts) & 0xF).astype(jnp.float32)
    w = (nib - 8.0) * srep_rows(s, k8, groups)[:, None, :]
    return w.reshape(k8 * 8, n).astype(jnp.bfloat16)

  def dequant_planes(q, s, k8, groups):
    n = q.shape[-1]
    srep = srep_rows(s, k8, groups)
    sbits = ((jax.lax.bitcast_convert_type(srep, jnp.int32) + 0x8000) >> 16)
    spk = pltpu.bitcast((sbits << 16) | sbits, jnp.bfloat16)
    planes = []
    for j in range(4):
      lo = (q >> (8 * j)) & 0xF
      if j < 2:
        hi = (q << (12 - 8 * j)) & 0xF0000
      else:
        hi = (q >> (8 * j - 12)) & 0xF0000
      w = lo | hi | 0x43004300
      wb = pltpu.bitcast(w, jnp.bfloat16)
      planes.append((wb - jnp.bfloat16(136.0)) * spk)
    return jnp.concatenate(planes, axis=0)

  x = x_ref[...]
  wg = dequant_planes(wg_ref[0], sg_ref[0], _D // 8, _D // _GROUP)
  hg = jnp.dot(x, wg, preferred_element_type=jnp.float32)
  wu = dequant_planes(wu_ref[0], su_ref[0], _D // 8, _D // _GROUP)
  hu = jnp.dot(x, wu, preferred_element_type=jnp.float32)

  we = jnp.sum(jnp.where(ids_ref[...] == e, tw_ref[...], 0.0), axis=1,
               keepdims=True)
  act = (hg * jax.nn.sigmoid(hg) * hu * we).astype(jnp.bfloat16)
  act = jnp.dot(act, p_ref[...], preferred_element_type=jnp.float32)
  act = act.astype(jnp.bfloat16)

  w2 = dequant_planes(w2_ref[0], s2_ref[0, 0], _FB // 8, _FB // _GROUP)
  o = jnp.dot(act, w2, preferred_element_type=jnp.float32)

  @pl.when((e == 0) & (j == 0))
  def _init():
    out_ref[...] = jnp.zeros_like(out_ref)

  out_ref[...] += o


@jax.jit
def kernel(x, topk_ids, topk_weights, w13_qweight, w13_scales, w2_qweight,
           w2_scales):
  jblocks = _F // _FB
  grid = (_E, jblocks)

  out = pl.pallas_call(
      _moe_kernel,
      grid=grid,
      in_specs=[
          pl.BlockSpec((_T, 2), lambda e, j: (0, 0)),
          pl.BlockSpec((_T, 2), lambda e, j: (0, 0)),
          pl.BlockSpec((_T, _D), lambda e, j: (0, 0)),
          pl.BlockSpec((_FB, _FB), lambda e, j: (0, 0)),
          pl.BlockSpec((1, _D // 8, _FB), lambda e, j: (e, 0, j)),
          pl.BlockSpec((1, _D // _GROUP, _FB), lambda e, j: (e, 0, j)),
          pl.BlockSpec((1, _D // 8, _FB), lambda e, j: (e, 0, j + jblocks)),
          pl.BlockSpec((1, _D // _GROUP, _FB), lambda e, j: (e, 0, j + jblocks)),
          pl.BlockSpec((1, _FB // 8, _D), lambda e, j: (e, j, 0)),
          pl.BlockSpec((1, 1, _FB // _GROUP, _D), lambda e, j: (e, j, 0, 0)),
      ],
      out_specs=pl.BlockSpec((_T, _D), lambda e, j: (0, 0)),
      out_shape=jax.ShapeDtypeStruct((_T, _D), jnp.float32),
      compiler_params=pltpu.CompilerParams(
          dimension_semantics=("arbitrary", "arbitrary"),
      ),
  )(
      topk_ids.astype(jnp.int32),
      topk_weights,
      x.reshape(_T, _D // 8, 4, 2).transpose(0, 2, 1, 3).reshape(_T, _D)
      .astype(jnp.bfloat16),
      (jnp.arange(_FB)[:, None]
       == (((jnp.arange(_FB)[None, :] % (_FB // 4)) // 2) * 8
           + 2 * (jnp.arange(_FB)[None, :] // (_FB // 4))
           + jnp.arange(_FB)[None, :] % 2)).astype(jnp.bfloat16),
      w13_qweight,
      w13_scales,
      w13_qweight,
      w13_scales,
      w2_qweight,
      w2_scales.reshape(_E, jblocks, _FB // _GROUP, _D),
  )
  return out


# speedup vs baseline: 4.2907x; 1.0343x over previous
import jax
import jax.numpy as jnp
from jax.experimental import pallas as pl
from jax.experimental.pallas import tpu as pltpu

_E = 8
_D = 1024
_F = 2048
_T = 128
_GROUP = 128
_FB = 1024


def _moe_kernel(ids_ref, tw_ref, x_ref, p_ref, wg_ref, sg_ref,
                wu_ref, su_ref, w2_ref, s2_ref, out_ref):
  e = pl.program_id(0)
  j = pl.program_id(1)

  shifts = (jnp.arange(8, dtype=jnp.int32) * 4)[None, :, None]

  def srep_rows(s, k8, groups):
    n = s.shape[-1]
    srep = jnp.broadcast_to(s[:, None, :], (groups, _GROUP // 8, n))
    return srep.reshape(k8, n)

  def dequant(q, s, k8, groups):
    n = q.shape[-1]
    nib = ((q[:, None, :] >> shifts) & 0xF).astype(jnp.float32)
    w = (nib - 8.0) * srep_rows(s, k8, groups)[:, None, :]
    return w.reshape(k8 * 8, n).astype(jnp.bfloat16)

  def dequant_planes(q, s, k8, groups):
    n = q.shape[-1]
    sbits = ((jax.lax.bitcast_convert_type(s, jnp.int32) + 0x8000) >> 16)
    sword = srep_rows((sbits << 16) | sbits, k8, groups)
    spk = pltpu.bitcast(sword, jnp.bfloat16)
    planes = []
    for j in range(4):
      lo = (q >> (8 * j)) & 0xF
      if j < 2:
        hi = (q << (12 - 8 * j)) & 0xF0000
      else:
        hi = (q >> (8 * j - 12)) & 0xF0000
      w = lo | hi | 0x43004300
      wb = pltpu.bitcast(w, jnp.bfloat16)
      planes.append((wb - jnp.bfloat16(136.0)) * spk)
    return jnp.concatenate(planes, axis=0)

  x = x_ref[...]
  wg = dequant_planes(wg_ref[0], sg_ref[0], _D // 8, _D // _GROUP)
  hg = jnp.dot(x, wg, preferred_element_type=jnp.float32)
  wu = dequant_planes(wu_ref[0], su_ref[0], _D // 8, _D // _GROUP)
  hu = jnp.dot(x, wu, preferred_element_type=jnp.float32)

  we = jnp.sum(jnp.where(ids_ref[...] == e, tw_ref[...], 0.0), axis=1,
               keepdims=True)
  act = (hg * jax.nn.sigmoid(hg) * hu * we).astype(jnp.bfloat16)
  act = jnp.dot(act, p_ref[...], preferred_element_type=jnp.float32)
  act = act.astype(jnp.bfloat16)

  w2 = dequant_planes(w2_ref[0], s2_ref[0, 0], _FB // 8, _FB // _GROUP)
  o = jnp.dot(act, w2, preferred_element_type=jnp.float32)

  @pl.when((e == 0) & (j == 0))
  def _init():
    out_ref[...] = jnp.zeros_like(out_ref)

  out_ref[...] += o


@jax.jit
def kernel(x, topk_ids, topk_weights, w13_qweight, w13_scales, w2_qweight,
           w2_scales):
  jblocks = _F // _FB
  grid = (_E, jblocks)

  out = pl.pallas_call(
      _moe_kernel,
      grid=grid,
      in_specs=[
          pl.BlockSpec((_T, 2), lambda e, j: (0, 0)),
          pl.BlockSpec((_T, 2), lambda e, j: (0, 0)),
          pl.BlockSpec((_T, _D), lambda e, j: (0, 0)),
          pl.BlockSpec((_FB, _FB), lambda e, j: (0, 0)),
          pl.BlockSpec((1, _D // 8, _FB), lambda e, j: (e, 0, j)),
          pl.BlockSpec((1, _D // _GROUP, _FB), lambda e, j: (e, 0, j)),
          pl.BlockSpec((1, _D // 8, _FB), lambda e, j: (e, 0, j + jblocks)),
          pl.BlockSpec((1, _D // _GROUP, _FB), lambda e, j: (e, 0, j + jblocks)),
          pl.BlockSpec((1, _FB // 8, _D), lambda e, j: (e, j, 0)),
          pl.BlockSpec((1, 1, _FB // _GROUP, _D), lambda e, j: (e, j, 0, 0)),
      ],
      out_specs=pl.BlockSpec((_T, _D), lambda e, j: (0, 0)),
      out_shape=jax.ShapeDtypeStruct((_T, _D), jnp.float32),
      compiler_params=pltpu.CompilerParams(
          dimension_semantics=("arbitrary", "arbitrary"),
      ),
  )(
      topk_ids.astype(jnp.int32),
      topk_weights,
      x.reshape(_T, _D // 8, 4, 2).transpose(0, 2, 1, 3).reshape(_T, _D)
      .astype(jnp.bfloat16),
      (jnp.arange(_FB)[:, None]
       == (((jnp.arange(_FB)[None, :] % (_FB // 4)) // 2) * 8
           + 2 * (jnp.arange(_FB)[None, :] // (_FB // 4))
           + jnp.arange(_FB)[None, :] % 2)).astype(jnp.bfloat16),
      w13_qweight,
      w13_scales,
      w13_qweight,
      w13_scales,
      w2_qweight,
      w2_scales.reshape(_E, jblocks, _FB // _GROUP, _D),
  )
  return out


# speedup vs baseline: 4.4851x; 1.0453x over previous
import jax
import jax.numpy as jnp
import numpy as np
from jax.experimental import pallas as pl
from jax.experimental.pallas import tpu as pltpu

_E = 8
_D = 1024
_F = 2048
_T = 128
_GROUP = 128
_FB = 1024

_P_ACT = (np.arange(_FB)[:, None]
          == (((np.arange(_FB)[None, :] % (_FB // 4)) // 2) * 8
              + 2 * (np.arange(_FB)[None, :] // (_FB // 4))
              + np.arange(_FB)[None, :] % 2)).astype(jnp.bfloat16)


def _moe_kernel(ids_ref, tw_ref, x_ref, p_ref, wg_ref, sg_ref,
                wu_ref, su_ref, w2_ref, s2_ref, out_ref):
  e = pl.program_id(0)
  j = pl.program_id(1)

  shifts = (jnp.arange(8, dtype=jnp.int32) * 4)[None, :, None]

  def srep_rows(s, k8, groups):
    n = s.shape[-1]
    srep = jnp.broadcast_to(s[:, None, :], (groups, _GROUP // 8, n))
    return srep.reshape(k8, n)

  def dequant(q, s, k8, groups):
    n = q.shape[-1]
    nib = ((q[:, None, :] >> shifts) & 0xF).astype(jnp.float32)
    w = (nib - 8.0) * srep_rows(s, k8, groups)[:, None, :]
    return w.reshape(k8 * 8, n).astype(jnp.bfloat16)

  def dequant_planes(q, s, k8, groups):
    n = q.shape[-1]
    sbits = ((jax.lax.bitcast_convert_type(s, jnp.int32) + 0x8000) >> 16)
    sword = srep_rows((sbits << 16) | sbits, k8, groups)
    spk = pltpu.bitcast(sword, jnp.bfloat16)
    planes = []
    for j in range(4):
      lo = (q >> (8 * j)) & 0xF
      if j < 2:
        hi = (q << (12 - 8 * j)) & 0xF0000
      else:
        hi = (q >> (8 * j - 12)) & 0xF0000
      w = lo | hi | 0x43004300
      wb = pltpu.bitcast(w, jnp.bfloat16)
      planes.append((wb - jnp.bfloat16(136.0)) * spk)
    return jnp.concatenate(planes, axis=0)

  x = x_ref[...]
  wg = dequant_planes(wg_ref[0], sg_ref[0], _D // 8, _D // _GROUP)
  hg = jnp.dot(x, wg, preferred_element_type=jnp.float32)
  wu = dequant_planes(wu_ref[0], su_ref[0], _D // 8, _D // _GROUP)
  hu = jnp.dot(x, wu, preferred_element_type=jnp.float32)

  we = jnp.sum(jnp.where(ids_ref[...] == e, tw_ref[...], 0.0), axis=1,
               keepdims=True)
  act = (hg * jax.nn.sigmoid(hg) * hu * we).astype(jnp.bfloat16)
  act = jnp.dot(act, p_ref[...], preferred_element_type=jnp.float32)
  act = act.astype(jnp.bfloat16)

  w2 = dequant_planes(w2_ref[0], s2_ref[0, 0], _FB // 8, _FB // _GROUP)
  o = jnp.dot(act, w2, preferred_element_type=jnp.float32)

  @pl.when((e == 0) & (j == 0))
  def _init():
    out_ref[...] = jnp.zeros_like(out_ref)

  out_ref[...] += o


@jax.jit
def kernel(x, topk_ids, topk_weights, w13_qweight, w13_scales, w2_qweight,
           w2_scales):
  jblocks = _F // _FB
  grid = (_E, jblocks)

  out = pl.pallas_call(
      _moe_kernel,
      grid=grid,
      in_specs=[
          pl.BlockSpec((_T, 2), lambda e, j: (0, 0)),
          pl.BlockSpec((_T, 2), lambda e, j: (0, 0)),
          pl.BlockSpec((_T, _D), lambda e, j: (0, 0)),
          pl.BlockSpec((_FB, _FB), lambda e, j: (0, 0)),
          pl.BlockSpec((1, _D // 8, _FB), lambda e, j: (e, 0, j)),
          pl.BlockSpec((1, _D // _GROUP, _FB), lambda e, j: (e, 0, j)),
          pl.BlockSpec((1, _D // 8, _FB), lambda e, j: (e, 0, j + jblocks)),
          pl.BlockSpec((1, _D // _GROUP, _FB), lambda e, j: (e, 0, j + jblocks)),
          pl.BlockSpec((1, _FB // 8, _D), lambda e, j: (e, j, 0)),
          pl.BlockSpec((1, 1, _FB // _GROUP, _D), lambda e, j: (e, j, 0, 0)),
      ],
      out_specs=pl.BlockSpec((_T, _D), lambda e, j: (0, 0)),
      out_shape=jax.ShapeDtypeStruct((_T, _D), jnp.float32),
      compiler_params=pltpu.CompilerParams(
          dimension_semantics=("arbitrary", "arbitrary"),
      ),
  )(
      topk_ids.astype(jnp.int32),
      topk_weights,
      x.reshape(_T, _D // 8, 4, 2).transpose(0, 2, 1, 3).reshape(_T, _D)
      .astype(jnp.bfloat16),
      _P_ACT,
      w13_qweight,
      w13_scales,
      w13_qweight,
      w13_scales,
      w2_qweight,
      w2_scales.reshape(_E, jblocks, _FB // _GROUP, _D),
  )
  return out
